# baseline (speedup 1.0000x reference)
from math import comb

import jax
import jax.numpy as jnp
from jax import lax
from jax.experimental import pallas as pl
from jax.experimental.pallas import tpu as pltpu
from jax.experimental.pallas import tpu_sc as plsc

NUM_CORES = 2
NUM_SUBCORES = 16
LANES = 16
LUT_N = 512
GROUP = 4
QROWS = 128
SROWS = 32
NSEC = QROWS // SROWS
VPI = 2
UNROLL = 16


def _build_lut(control_points, u, dtype):
    t = jnp.linspace(-1.0, 1.0, LUT_N, dtype=dtype)
    cpy = control_points[:, 1]
    n = control_points.shape[0] - 1
    bern = jnp.stack(
        [comb(n, k) * t**k * (1.0 - t) ** (n - k) for k in range(n + 1)], axis=0
    )
    fp = jnp.clip(cpy @ bern, -1.0, 1.0)
    slope = (fp[1:] - fp[:-1]) / (t[1:] - t[:-1])
    a_seg = fp[:-1] - slope * t[:-1]
    a = jnp.concatenate([a_seg[:1], a_seg])
    b = jnp.concatenate([slope[:1], slope])
    sign = jnp.where(u[0] > 0.5, 1.0, -1.0).astype(dtype)
    return jnp.repeat(a * sign, LANES), jnp.repeat(b * sign, LANES)


def _sc_body(
    img_hbm, a_hbm, b_hbm, out_hbm,
    qbuf, abuf, bbuf, mmstage, gbuf, shared, isems, osems,
):
    n_imgs, _, height, width = img_hbm.shape
    imgs_per_core = n_imgs // NUM_CORES
    n_rounds = imgs_per_core // GROUP
    sec_px = SROWS * width

    cid = lax.axis_index("c")
    sid = lax.axis_index("s")
    grp = sid // GROUP
    mem = sid % GROUP
    row_base = mem * QROWS
    img_base = cid * imgs_per_core + grp

    pltpu.sync_copy(a_hbm, abuf)
    pltpu.sync_copy(b_hbm, bbuf)

    lane = lax.iota(jnp.int32, LANES)

    def interp(x):
        idx = jnp.clip((x * 255.5 + 256.5).astype(jnp.int32), 1, LUT_N - 1)
        cidx = lax.shift_left(idx, 4) + lane
        av = plsc.load_gather(abuf, [cidx])
        bv = plsc.load_gather(bbuf, [cidx])
        return av + bv * x

    def start_in(img, c):
        pltpu.async_copy(
            img_hbm.at[img, 0, pl.ds(row_base + c * SROWS, SROWS)],
            qbuf.at[pl.ds(c * SROWS, SROWS)],
            isems.at[c],
        )

    def wait_in(c):
        pltpu.make_async_copy(
            img_hbm.at[0, 0, pl.ds(0, SROWS)],
            qbuf.at[pl.ds(c * SROWS, SROWS)],
            isems.at[c],
        ).wait()

    def start_out(img, c):
        pltpu.async_copy(
            qbuf.at[pl.ds(c * SROWS, SROWS)],
            out_hbm.at[img, 0, pl.ds(row_base + c * SROWS, SROWS)],
            osems.at[c],
        )

    def wait_out(c):
        pltpu.make_async_copy(
            qbuf.at[pl.ds(c * SROWS, SROWS)],
            out_hbm.at[0, 0, pl.ds(0, SROWS)],
            osems.at[c],
        ).wait()

    def round_body(r, carry):
        img = img_base + r * GROUP

        for c in range(NSEC):
            @pl.when(r > 0)
            def _():
                wait_out(c)

            start_in(img, c)

        acc = (
            jnp.full((LANES,), jnp.inf, jnp.float32),
            jnp.full((LANES,), -jnp.inf, jnp.float32),
        ) * VPI
        for c in range(NSEC):
            wait_in(c)
            row0 = c * SROWS

            def stepa(i, carry_a):
                row = row0 + lax.shift_right_logical(i, 9)
                col = lax.bitwise_and(i, width - 1)
                out = []
                for k in range(VPI):
                    cmn, cmx = carry_a[2 * k], carry_a[2 * k + 1]
                    x = qbuf[row, pl.ds(col + k * LANES, LANES)]
                    v = interp(x)
                    qbuf[row, pl.ds(col + k * LANES, LANES)] = v
                    out += [jnp.minimum(cmn, v), jnp.maximum(cmx, v)]
                return tuple(out)

            acc = plsc.parallel_loop(
                0, sec_px, VPI * LANES, unroll=UNROLL, carry=acc
            )(stepa)

        mn, mx = acc[0], acc[1]
        for k in range(1, VPI):
            mn = jnp.minimum(mn, acc[2 * k])
            mx = jnp.maximum(mx, acc[2 * k + 1])

        mmstage[pl.ds(0, LANES)] = mn
        mmstage[pl.ds(LANES, LANES)] = mx
        pltpu.sync_copy(mmstage, shared.at[sid])
        plsc.subcore_barrier()
        pltpu.sync_copy(shared, gbuf)
        gmn = gbuf[grp * GROUP, pl.ds(0, LANES)]
        gmx = gbuf[grp * GROUP, pl.ds(LANES, LANES)]
        for j in range(1, GROUP):
            gmn = jnp.minimum(gmn, gbuf[grp * GROUP + j, pl.ds(0, LANES)])
            gmx = jnp.maximum(gmx, gbuf[grp * GROUP + j, pl.ds(LANES, LANES)])
        plsc.subcore_barrier()

        mnv = jnp.full((LANES,), jnp.min(gmn), jnp.float32)
        mxv = jnp.full((LANES,), jnp.max(gmx), jnp.float32)
        scv = 2.0 / (mxv - mnv)
        ofv = -mnv * scv - 1.0

        for c in range(NSEC):
            row0 = c * SROWS

            def stepb(i, carry_b):
                row = row0 + lax.shift_right_logical(i, 9)
                col = lax.bitwise_and(i, width - 1)
                for k in range(VPI):
                    v = qbuf[row, pl.ds(col + k * LANES, LANES)]
                    qbuf[row, pl.ds(col + k * LANES, LANES)] = v * scv + ofv
                return carry_b

            plsc.parallel_loop(
                0, sec_px, VPI * LANES, unroll=UNROLL, carry=jnp.int32(0)
            )(stepb)
            start_out(img, c)
        return carry

    lax.fori_loop(0, n_rounds, round_body, jnp.int32(0))
    for c in range(NSEC):
        wait_out(c)


def kernel(image, control_points, u):
    a_lut, b_lut = _build_lut(control_points, u, image.dtype)

    mesh = plsc.VectorSubcoreMesh(
        core_axis_name="c",
        subcore_axis_name="s",
        num_cores=NUM_CORES,
        num_subcores=NUM_SUBCORES,
    )
    return pl.kernel(
        _sc_body,
        out_type=jax.ShapeDtypeStruct(image.shape, jnp.float32),
        mesh=mesh,
        compiler_params=pltpu.CompilerParams(needs_layout_passes=False),
        scratch_types=[
            pltpu.VMEM((QROWS, 512), jnp.float32),
            pltpu.VMEM((LUT_N * LANES,), jnp.float32),
            pltpu.VMEM((LUT_N * LANES,), jnp.float32),
            pltpu.VMEM((128,), jnp.float32),
            pltpu.VMEM((NUM_SUBCORES, 128), jnp.float32),
            pltpu.VMEM_SHARED((NUM_SUBCORES, 128), jnp.float32),
            pltpu.SemaphoreType.DMA((NSEC,)),
            pltpu.SemaphoreType.DMA((NSEC,)),
        ],
    )(image, a_lut, b_lut)


# speedup vs baseline: 1.2398x; 1.2398x over previous
from math import comb

import jax
import jax.numpy as jnp
from jax import lax
from jax.experimental import pallas as pl
from jax.experimental.pallas import tpu as pltpu
from jax.experimental.pallas import tpu_sc as plsc

NUM_CORES = 2
NUM_SUBCORES = 16
LANES = 16
LUT_N = 512
GROUP = 4
QROWS = 128
SROWS = 64
NSEC = QROWS // SROWS
VPI = 2
UNROLL = 8


def _build_lut(control_points, u, dtype):
    t = jnp.linspace(-1.0, 1.0, LUT_N, dtype=dtype)
    cpy = control_points[:, 1]
    n = control_points.shape[0] - 1
    bern = jnp.stack(
        [comb(n, k) * t**k * (1.0 - t) ** (n - k) for k in range(n + 1)], axis=0
    )
    fp = jnp.clip(cpy @ bern, -1.0, 1.0)
    slope = (fp[1:] - fp[:-1]) / (t[1:] - t[:-1])
    a_seg = fp[:-1] - slope * t[:-1]
    a = jnp.concatenate([a_seg[:1], a_seg])
    b = jnp.concatenate([slope[:1], slope])
    sign = jnp.where(u[0] > 0.5, 1.0, -1.0).astype(dtype)
    return jnp.repeat(a * sign, LANES), jnp.repeat(b * sign, LANES)


def _sc_body(
    img_hbm, a_hbm, b_hbm, out_hbm,
    qbuf, abuf, bbuf, mmstage, gbuf, shared, isems, osems,
):
    n_imgs, _, height, width = img_hbm.shape
    imgs_per_core = n_imgs // NUM_CORES
    n_rounds = imgs_per_core // GROUP
    sec_px = SROWS * width

    cid = lax.axis_index("c")
    sid = lax.axis_index("s")
    grp = sid // GROUP
    mem = sid % GROUP
    row_base = mem * QROWS
    img_base = cid * imgs_per_core + grp

    pltpu.sync_copy(a_hbm, abuf)
    pltpu.sync_copy(b_hbm, bbuf)

    lane = lax.iota(jnp.int32, LANES)

    def interp(x):
        idx = jnp.clip((x * 255.5 + 256.5).astype(jnp.int32), 1, LUT_N - 1)
        cidx = lax.shift_left(idx, 4) + lane
        av = plsc.load_gather(abuf, [cidx])
        bv = plsc.load_gather(bbuf, [cidx])
        return av + bv * x

    def start_in(img, c):
        pltpu.async_copy(
            img_hbm.at[img, 0, pl.ds(row_base + c * SROWS, SROWS)],
            qbuf.at[pl.ds(c * SROWS, SROWS)],
            isems.at[c],
        )

    def wait_in(c):
        pltpu.make_async_copy(
            img_hbm.at[0, 0, pl.ds(0, SROWS)],
            qbuf.at[pl.ds(c * SROWS, SROWS)],
            isems.at[c],
        ).wait()

    def start_out(img, c):
        pltpu.async_copy(
            qbuf.at[pl.ds(c * SROWS, SROWS)],
            out_hbm.at[img, 0, pl.ds(row_base + c * SROWS, SROWS)],
            osems.at[c],
        )

    def wait_out(c):
        pltpu.make_async_copy(
            qbuf.at[pl.ds(c * SROWS, SROWS)],
            out_hbm.at[0, 0, pl.ds(0, SROWS)],
            osems.at[c],
        ).wait()

    def round_body(r, carry):
        img = img_base + r * GROUP

        for c in range(NSEC):
            @pl.when(r > 0)
            def _():
                wait_out(c)

            start_in(img, c)

        acc = (
            jnp.full((LANES,), jnp.inf, jnp.float32),
            jnp.full((LANES,), -jnp.inf, jnp.float32),
        ) * VPI
        for c in range(NSEC):
            wait_in(c)
            row0 = c * SROWS

            def stepa(i, carry_a):
                row = row0 + lax.shift_right_logical(i, 9)
                col = lax.bitwise_and(i, width - 1)
                out = []
                for k in range(VPI):
                    cmn, cmx = carry_a[2 * k], carry_a[2 * k + 1]
                    x = qbuf[row, pl.ds(col + k * LANES, LANES)]
                    v = interp(x)
                    qbuf[row, pl.ds(col + k * LANES, LANES)] = v
                    out += [jnp.minimum(cmn, v), jnp.maximum(cmx, v)]
                return tuple(out)

            acc = plsc.parallel_loop(
                0, sec_px, VPI * LANES, unroll=UNROLL, carry=acc
            )(stepa)

        mn, mx = acc[0], acc[1]
        for k in range(1, VPI):
            mn = jnp.minimum(mn, acc[2 * k])
            mx = jnp.maximum(mx, acc[2 * k + 1])

        mmstage[pl.ds(0, LANES)] = mn
        mmstage[pl.ds(LANES, LANES)] = mx
        pltpu.sync_copy(mmstage, shared.at[sid])
        plsc.subcore_barrier()
        pltpu.sync_copy(shared, gbuf)
        gmn = gbuf[grp * GROUP, pl.ds(0, LANES)]
        gmx = gbuf[grp * GROUP, pl.ds(LANES, LANES)]
        for j in range(1, GROUP):
            gmn = jnp.minimum(gmn, gbuf[grp * GROUP + j, pl.ds(0, LANES)])
            gmx = jnp.maximum(gmx, gbuf[grp * GROUP + j, pl.ds(LANES, LANES)])
        plsc.subcore_barrier()

        mnv = jnp.full((LANES,), jnp.min(gmn), jnp.float32)
        mxv = jnp.full((LANES,), jnp.max(gmx), jnp.float32)
        scv = 2.0 / (mxv - mnv)
        ofv = -mnv * scv - 1.0

        for c in range(NSEC):
            row0 = c * SROWS

            def stepb(i, carry_b):
                row = row0 + lax.shift_right_logical(i, 9)
                col = lax.bitwise_and(i, width - 1)
                for k in range(VPI):
                    v = qbuf[row, pl.ds(col + k * LANES, LANES)]
                    qbuf[row, pl.ds(col + k * LANES, LANES)] = v * scv + ofv
                return carry_b

            plsc.parallel_loop(
                0, sec_px, VPI * LANES, unroll=UNROLL, carry=jnp.int32(0)
            )(stepb)
            start_out(img, c)
        return carry

    lax.fori_loop(0, n_rounds, round_body, jnp.int32(0))
    for c in range(NSEC):
        wait_out(c)


def kernel(image, control_points, u):
    a_lut, b_lut = _build_lut(control_points, u, image.dtype)

    mesh = plsc.VectorSubcoreMesh(
        core_axis_name="c",
        subcore_axis_name="s",
        num_cores=NUM_CORES,
        num_subcores=NUM_SUBCORES,
    )
    return pl.kernel(
        _sc_body,
        out_type=jax.ShapeDtypeStruct(image.shape, jnp.float32),
        mesh=mesh,
        compiler_params=pltpu.CompilerParams(needs_layout_passes=False),
        scratch_types=[
            pltpu.VMEM((QROWS, 512), jnp.float32),
            pltpu.VMEM((LUT_N * LANES,), jnp.float32),
            pltpu.VMEM((LUT_N * LANES,), jnp.float32),
            pltpu.VMEM((128,), jnp.float32),
            pltpu.VMEM((NUM_SUBCORES, 128), jnp.float32),
            pltpu.VMEM_SHARED((NUM_SUBCORES, 128), jnp.float32),
            pltpu.SemaphoreType.DMA((NSEC,)),
            pltpu.SemaphoreType.DMA((NSEC,)),
        ],
    )(image, a_lut, b_lut)


# speedup vs baseline: 1.2402x; 1.0003x over previous
from math import comb

import jax
import jax.numpy as jnp
from jax import lax
from jax.experimental import pallas as pl
from jax.experimental.pallas import tpu as pltpu
from jax.experimental.pallas import tpu_sc as plsc

NUM_CORES = 2
NUM_SUBCORES = 16
LANES = 16
LUT_N = 512
GROUP = 4
QROWS = 128
SROWS = 64
NSEC = QROWS // SROWS
VPI = 2
UNROLL = 8


def _build_lut(control_points, u, dtype):
    t = jnp.linspace(-1.0, 1.0, LUT_N, dtype=dtype)
    cpy = control_points[:, 1]
    n = control_points.shape[0] - 1
    bern = jnp.stack(
        [comb(n, k) * t**k * (1.0 - t) ** (n - k) for k in range(n + 1)], axis=0
    )
    fp = jnp.clip(cpy @ bern, -1.0, 1.0)
    slope = (fp[1:] - fp[:-1]) / (t[1:] - t[:-1])
    a_seg = fp[:-1] - slope * t[:-1]
    a = jnp.concatenate([a_seg[:1], a_seg])
    b = jnp.concatenate([slope[:1], slope])
    sign = jnp.where(u[0] > 0.5, 1.0, -1.0).astype(dtype)
    return jnp.repeat(a * sign, LANES), jnp.repeat(b * sign, LANES)


def _sc_body(
    img_hbm, a_hbm, b_hbm, out_hbm,
    qbuf, abuf, bbuf, mmstage, gbuf, shared, isems, osems,
):
    n_imgs, _, height, width = img_hbm.shape
    imgs_per_core = n_imgs // NUM_CORES
    n_rounds = imgs_per_core // GROUP
    sec_px = SROWS * width

    cid = lax.axis_index("c")
    sid = lax.axis_index("s")
    grp = sid // GROUP
    mem = sid % GROUP
    row_base = mem * QROWS
    img_base = cid * imgs_per_core + grp

    pltpu.sync_copy(a_hbm, abuf)
    pltpu.sync_copy(b_hbm, bbuf)

    lane = lax.iota(jnp.int32, LANES)

    def interp(x):
        idx = jnp.clip((x * 255.5 + 256.5).astype(jnp.int32), 1, LUT_N - 1)
        cidx = lax.shift_left(idx, 4) + lane
        av = plsc.load_gather(abuf, [cidx])
        bv = plsc.load_gather(bbuf, [cidx])
        return av + bv * x

    def start_in(img, c):
        pltpu.async_copy(
            img_hbm.at[img, 0, pl.ds(row_base + c * SROWS, SROWS)],
            qbuf.at[pl.ds(c * SROWS, SROWS)],
            isems.at[c],
        )

    def wait_in(c):
        pltpu.make_async_copy(
            img_hbm.at[0, 0, pl.ds(0, SROWS)],
            qbuf.at[pl.ds(c * SROWS, SROWS)],
            isems.at[c],
        ).wait()

    def start_out(img, c):
        pltpu.async_copy(
            qbuf.at[pl.ds(c * SROWS, SROWS)],
            out_hbm.at[img, 0, pl.ds(row_base + c * SROWS, SROWS)],
            osems.at[c],
        )

    def wait_out(c):
        pltpu.make_async_copy(
            qbuf.at[pl.ds(c * SROWS, SROWS)],
            out_hbm.at[0, 0, pl.ds(0, SROWS)],
            osems.at[c],
        ).wait()

    def round_body(r, carry):
        img = img_base + r * GROUP

        @pl.when(r > 0)
        def _():
            wait_out(0)

        start_in(img, 0)

        acc = (
            jnp.full((LANES,), jnp.inf, jnp.float32),
            jnp.full((LANES,), -jnp.inf, jnp.float32),
        ) * VPI
        for c in range(NSEC):
            if c + 1 < NSEC:
                @pl.when(r > 0)
                def _():
                    wait_out(c + 1)

                start_in(img, c + 1)
            wait_in(c)
            row0 = c * SROWS

            def stepa(i, carry_a):
                row = row0 + lax.shift_right_logical(i, 9)
                col = lax.bitwise_and(i, width - 1)
                out = []
                for k in range(VPI):
                    cmn, cmx = carry_a[2 * k], carry_a[2 * k + 1]
                    x = qbuf[row, pl.ds(col + k * LANES, LANES)]
                    v = interp(x)
                    qbuf[row, pl.ds(col + k * LANES, LANES)] = v
                    out += [jnp.minimum(cmn, v), jnp.maximum(cmx, v)]
                return tuple(out)

            acc = plsc.parallel_loop(
                0, sec_px, VPI * LANES, unroll=UNROLL, carry=acc
            )(stepa)

        mn, mx = acc[0], acc[1]
        for k in range(1, VPI):
            mn = jnp.minimum(mn, acc[2 * k])
            mx = jnp.maximum(mx, acc[2 * k + 1])

        mmstage[pl.ds(0, LANES)] = mn
        mmstage[pl.ds(LANES, LANES)] = mx
        pltpu.sync_copy(mmstage, shared.at[sid])
        plsc.subcore_barrier()
        pltpu.sync_copy(shared, gbuf)
        gmn = gbuf[grp * GROUP, pl.ds(0, LANES)]
        gmx = gbuf[grp * GROUP, pl.ds(LANES, LANES)]
        for j in range(1, GROUP):
            gmn = jnp.minimum(gmn, gbuf[grp * GROUP + j, pl.ds(0, LANES)])
            gmx = jnp.maximum(gmx, gbuf[grp * GROUP + j, pl.ds(LANES, LANES)])
        plsc.subcore_barrier()

        mnv = jnp.full((LANES,), jnp.min(gmn), jnp.float32)
        mxv = jnp.full((LANES,), jnp.max(gmx), jnp.float32)
        scv = 2.0 / (mxv - mnv)
        ofv = -mnv * scv - 1.0

        for c in range(NSEC):
            row0 = c * SROWS

            def stepb(i, carry_b):
                row = row0 + lax.shift_right_logical(i, 9)
                col = lax.bitwise_and(i, width - 1)
                for k in range(VPI):
                    v = qbuf[row, pl.ds(col + k * LANES, LANES)]
                    qbuf[row, pl.ds(col + k * LANES, LANES)] = v * scv + ofv
                return carry_b

            plsc.parallel_loop(
                0, sec_px, VPI * LANES, unroll=UNROLL, carry=jnp.int32(0)
            )(stepb)
            start_out(img, c)
        return carry

    lax.fori_loop(0, n_rounds, round_body, jnp.int32(0))
    for c in range(NSEC):
        wait_out(c)


def kernel(image, control_points, u):
    a_lut, b_lut = _build_lut(control_points, u, image.dtype)

    mesh = plsc.VectorSubcoreMesh(
        core_axis_name="c",
        subcore_axis_name="s",
        num_cores=NUM_CORES,
        num_subcores=NUM_SUBCORES,
    )
    return pl.kernel(
        _sc_body,
        out_type=jax.ShapeDtypeStruct(image.shape, jnp.float32),
        mesh=mesh,
        compiler_params=pltpu.CompilerParams(needs_layout_passes=False),
        scratch_types=[
            pltpu.VMEM((QROWS, 512), jnp.float32),
            pltpu.VMEM((LUT_N * LANES,), jnp.float32),
            pltpu.VMEM((LUT_N * LANES,), jnp.float32),
            pltpu.VMEM((128,), jnp.float32),
            pltpu.VMEM((NUM_SUBCORES, 128), jnp.float32),
            pltpu.VMEM_SHARED((NUM_SUBCORES, 128), jnp.float32),
            pltpu.SemaphoreType.DMA((NSEC,)),
            pltpu.SemaphoreType.DMA((NSEC,)),
        ],
    )(image, a_lut, b_lut)
